# Initial kernel scaffold; baseline (speedup 1.0000x reference)
import jax
import jax.numpy as jnp
from jax.experimental import pallas as pl


def kernel(x, edge_index, top_features, W1, b1, W2, b2, G1, G2, fcW, fcb):
    raise NotImplementedError("write your pallas kernel here")


# speedup vs baseline: 25.1959x; 25.1959x over previous
import functools

import jax
import jax.numpy as jnp
from jax import lax
from jax.experimental import pallas as pl
from jax.experimental.pallas import tpu as pltpu
from jax.experimental.pallas import tpu_sc as plsc

N = 10000
E = 320000
D = 128
NC = 2
NS = 16
NW = NC * NS
EPW = E // NW
CH = 128
NFULL = EPW // CH
TAIL = EPW - NFULL * CH
NP = 10240
RPT = NP // NS
DEGW = D

_MESH = plsc.VectorSubcoreMesh(core_axis_name="c", subcore_axis_name="s")
_HIGH = jax.lax.Precision.HIGHEST


@functools.partial(
    pl.kernel,
    out_type=jax.ShapeDtypeStruct((NC, NP, D), jnp.float32),
    mesh=_MESH,
    scratch_types=[
        pltpu.VMEM_SHARED((NP, D), jnp.float32),
        pltpu.VMEM((CH,), jnp.int32),
        pltpu.VMEM((TAIL,), jnp.int32),
        pltpu.VMEM((CH, D), jnp.float32),
    ],
)
def _sc_degree(dst_hbm, ones_hbm, zeros_hbm, degp_hbm,
               acc_sh, idx_v, idxt_v, ones_v):
    cid = lax.axis_index("c")
    sid = lax.axis_index("s")
    w = cid * NS + sid
    pltpu.sync_copy(zeros_hbm, acc_sh.at[pl.ds(sid * RPT, RPT)])
    pltpu.sync_copy(ones_hbm, ones_v)
    plsc.subcore_barrier()

    def body(c, carry):
        base = w * EPW + c * CH
        pltpu.sync_copy(dst_hbm.at[pl.ds(base, CH)], idx_v)
        pltpu.sync_copy(ones_v, acc_sh.at[idx_v], add=True)
        return carry

    lax.fori_loop(0, NFULL, body, 0)
    base = w * EPW + NFULL * CH
    pltpu.sync_copy(dst_hbm.at[pl.ds(base, TAIL)], idxt_v)
    pltpu.sync_copy(ones_v.at[pl.ds(0, TAIL)], acc_sh.at[idxt_v], add=True)
    plsc.subcore_barrier()
    pltpu.sync_copy(acc_sh.at[pl.ds(sid * RPT, RPT)],
                    degp_hbm.at[cid, pl.ds(sid * RPT, RPT)])


@functools.partial(
    pl.kernel,
    out_type=jax.ShapeDtypeStruct((NC, NP, D), jnp.float32),
    mesh=_MESH,
    scratch_types=[
        pltpu.VMEM_SHARED((NP, D), jnp.float32),
        pltpu.VMEM((CH,), jnp.int32),
        pltpu.VMEM((CH,), jnp.int32),
        pltpu.VMEM((TAIL,), jnp.int32),
        pltpu.VMEM((TAIL,), jnp.int32),
        pltpu.VMEM((CH, D), jnp.float32),
        pltpu.VMEM((TAIL, D), jnp.float32),
        pltpu.SemaphoreType.DMA,
    ],
)
def _sc_segsum(y_hbm, src_hbm, dst_hbm, zeros_hbm, part_hbm,
               acc_sh, idxs_v, idxd_v, idxs_t, idxd_t, rows_v, rows_t, sem):
    cid = lax.axis_index("c")
    sid = lax.axis_index("s")
    w = cid * NS + sid
    pltpu.sync_copy(zeros_hbm, acc_sh.at[pl.ds(sid * RPT, RPT)])
    plsc.subcore_barrier()

    def body(c, carry):
        base = w * EPW + c * CH
        pltpu.sync_copy(src_hbm.at[pl.ds(base, CH)], idxs_v)
        pltpu.sync_copy(dst_hbm.at[pl.ds(base, CH)], idxd_v)
        pltpu.async_copy(y_hbm.at[idxs_v], rows_v, sem).wait()
        pltpu.sync_copy(rows_v, acc_sh.at[idxd_v], add=True)
        return carry

    lax.fori_loop(0, NFULL, body, 0)
    base = w * EPW + NFULL * CH
    pltpu.sync_copy(src_hbm.at[pl.ds(base, TAIL)], idxs_t)
    pltpu.sync_copy(dst_hbm.at[pl.ds(base, TAIL)], idxd_t)
    pltpu.async_copy(y_hbm.at[idxs_t], rows_t, sem).wait()
    pltpu.sync_copy(rows_t, acc_sh.at[idxd_t], add=True)
    plsc.subcore_barrier()
    pltpu.sync_copy(acc_sh.at[pl.ds(sid * RPT, RPT)],
                    part_hbm.at[cid, pl.ds(sid * RPT, RPT)])


_BLK = 1000
_GRID = N // _BLK


def _dis_of(d0, d1):
    deg = d0[0, :, 0:1] + d1[0, :, 0:1] + 1.0
    return lax.rsqrt(deg)


def _tc_scale_body(d0, d1, x_ref, y_ref):
    y_ref[...] = _dis_of(d0, d1) * x_ref[...]


def _gate_of(top_ref, g_ref):
    logits = lax.dot_general(top_ref[...], g_ref[...], (((1,), (1,)), ((), ())),
                             precision=_HIGH) * (1.0 / 101.0)
    m = jnp.max(logits, axis=-1, keepdims=True)
    e = jnp.exp(logits - m)
    return e / jnp.sum(e, axis=-1, keepdims=True)


def _moe_of(agg, gate, w_ref, b_ref):
    out = jnp.zeros(agg.shape, agg.dtype)
    for i in range(3):
        eo = lax.dot_general(agg, w_ref[i], (((1,), (1,)), ((), ())),
                             precision=_HIGH) + b_ref[i][None, :]
        out = out + gate[:, i][:, None] * jnp.maximum(eo, 0.0)
    return out


def _tc_layer_body(s0, s1, d0, d1, x_ref, top_ref, w_ref, b_ref, g_ref,
                   h_ref, y2_ref):
    dis = _dis_of(d0, d1)
    agg = dis * (s0[0] + s1[0]) + (dis * dis) * x_ref[...]
    gate = _gate_of(top_ref, g_ref)
    h = _moe_of(agg, gate, w_ref, b_ref)
    h_ref[...] = h
    y2_ref[...] = dis * h


def _tc_final_body(s0, s1, d0, d1, h1_ref, top_ref, w_ref, b_ref, g_ref,
                   fcw_ref, fcb_ref, out_ref):
    dis = _dis_of(d0, d1)
    agg = dis * (s0[0] + s1[0]) + (dis * dis) * h1_ref[...]
    gate = _gate_of(top_ref, g_ref)
    h2 = _moe_of(agg, gate, w_ref, b_ref)
    out_ref[...] = lax.dot_general(h2, fcw_ref[...], (((1,), (1,)), ((), ())),
                                   precision=_HIGH) + fcb_ref[...][None, :]


def _half_specs(width):
    return [
        pl.BlockSpec((1, _BLK, width), lambda i: (0, i, 0)),
        pl.BlockSpec((1, _BLK, width), lambda i: (1, i, 0)),
    ]


def _full(shape):
    nd = len(shape)
    return pl.BlockSpec(shape, lambda i, _nd=nd: (0,) * _nd)


_tc_scale = pl.pallas_call(
    _tc_scale_body,
    grid=(_GRID,),
    in_specs=_half_specs(DEGW) + [pl.BlockSpec((_BLK, D), lambda i: (i, 0))],
    out_specs=pl.BlockSpec((_BLK, D), lambda i: (i, 0)),
    out_shape=jax.ShapeDtypeStruct((N, D), jnp.float32),
)

_tc_layer = pl.pallas_call(
    _tc_layer_body,
    grid=(_GRID,),
    in_specs=(
        _half_specs(D) + _half_specs(DEGW)
        + [pl.BlockSpec((_BLK, D), lambda i: (i, 0)),
           pl.BlockSpec((_BLK, 4), lambda i: (i, 0)),
           _full((3, D, D)), _full((3, D)), _full((3, 4))]
    ),
    out_specs=[pl.BlockSpec((_BLK, D), lambda i: (i, 0))] * 2,
    out_shape=[jax.ShapeDtypeStruct((N, D), jnp.float32)] * 2,
)

_tc_final = pl.pallas_call(
    _tc_final_body,
    grid=(_GRID,),
    in_specs=(
        _half_specs(D) + _half_specs(DEGW)
        + [pl.BlockSpec((_BLK, D), lambda i: (i, 0)),
           pl.BlockSpec((_BLK, 4), lambda i: (i, 0)),
           _full((3, D, D)), _full((3, D)), _full((3, 4)),
           _full((D, D)), _full((D,))]
    ),
    out_specs=pl.BlockSpec((_BLK, D), lambda i: (i, 0)),
    out_shape=jax.ShapeDtypeStruct((N, D), jnp.float32),
)


def kernel(x, edge_index, top_features, W1, b1, W2, b2, G1, G2, fcW, fcb):
    src = edge_index[0]
    dst = edge_index[1]
    ones_rows = jnp.ones((CH, D), jnp.float32)
    zeros_rows = jnp.zeros((RPT, D), jnp.float32)

    degp = _sc_degree(dst, ones_rows, zeros_rows)
    y1 = _tc_scale(degp, degp, x)
    s1p = _sc_segsum(y1, src, dst, zeros_rows)
    h1, y2 = _tc_layer(s1p, s1p, degp, degp, x, top_features, W1, b1, G1)
    s2p = _sc_segsum(y2, src, dst, zeros_rows)
    return _tc_final(s2p, s2p, degp, degp, h1, top_features,
                     W2, b2, G2, fcW, fcb)


# speedup vs baseline: 36.0688x; 1.4315x over previous
import functools

import jax
import jax.numpy as jnp
from jax import lax
from jax.experimental import pallas as pl
from jax.experimental.pallas import tpu as pltpu
from jax.experimental.pallas import tpu_sc as plsc

N = 10000
E = 320000
D = 128
NC = 2
NS = 16
NW = NC * NS
EPW = E // NW
CH = 128
NFULL = EPW // CH
TAIL = EPW - NFULL * CH
NP = 10240
RPT = NP // NS
DEGW = D

_MESH = plsc.VectorSubcoreMesh(core_axis_name="c", subcore_axis_name="s")
_HIGH = jax.lax.Precision.HIGHEST


@functools.partial(
    pl.kernel,
    out_type=jax.ShapeDtypeStruct((NC, NP, D), jnp.float32),
    mesh=_MESH,
    scratch_types=[
        pltpu.VMEM_SHARED((NP, D), jnp.float32),
        pltpu.VMEM((CH,), jnp.int32),
        pltpu.VMEM((CH,), jnp.int32),
        pltpu.VMEM((TAIL,), jnp.int32),
        pltpu.VMEM((CH, D), jnp.float32),
        pltpu.SemaphoreType.DMA,
        pltpu.SemaphoreType.DMA,
    ],
)
def _sc_degree(dst_hbm, ones_hbm, zeros_hbm, degp_hbm,
               acc_sh, idx0_v, idx1_v, idxt_v, ones_v, sem0, sem1):
    cid = lax.axis_index("c")
    sid = lax.axis_index("s")
    w = cid * NS + sid
    idx = (idx0_v, idx1_v)
    sems = (sem0, sem1)
    pltpu.sync_copy(zeros_hbm, acc_sh.at[pl.ds(sid * RPT, RPT)])
    pltpu.sync_copy(ones_hbm, ones_v)
    plsc.subcore_barrier()

    def group(g, carry):
        for j in (0, 1):
            c = 2 * g + j

            @pl.when(g > 0)
            def _():
                pltpu.make_async_copy(ones_v, acc_sh.at[idx[j]],
                                      sems[j]).wait()

            pltpu.sync_copy(dst_hbm.at[pl.ds(w * EPW + c * CH, CH)], idx[j])
            pltpu.async_copy(ones_v, acc_sh.at[idx[j]], sems[j], add=True)
        return carry

    lax.fori_loop(0, NFULL // 2, group, 0)
    for j in (0, 1):
        pltpu.make_async_copy(ones_v, acc_sh.at[idx[j]], sems[j]).wait()
    base = w * EPW + NFULL * CH
    pltpu.sync_copy(dst_hbm.at[pl.ds(base, TAIL)], idxt_v)
    pltpu.sync_copy(ones_v.at[pl.ds(0, TAIL)], acc_sh.at[idxt_v], add=True)
    plsc.subcore_barrier()
    pltpu.sync_copy(acc_sh.at[pl.ds(sid * RPT, RPT)],
                    degp_hbm.at[cid, pl.ds(sid * RPT, RPT)])


@functools.partial(
    pl.kernel,
    out_type=jax.ShapeDtypeStruct((NC, NP, D), jnp.float32),
    mesh=_MESH,
    scratch_types=[
        pltpu.VMEM_SHARED((NP, D), jnp.float32),
        pltpu.VMEM((CH,), jnp.int32),
        pltpu.VMEM((CH,), jnp.int32),
        pltpu.VMEM((CH,), jnp.int32),
        pltpu.VMEM((CH,), jnp.int32),
        pltpu.VMEM((TAIL,), jnp.int32),
        pltpu.VMEM((TAIL,), jnp.int32),
        pltpu.VMEM((CH, D), jnp.float32),
        pltpu.VMEM((CH, D), jnp.float32),
        pltpu.VMEM((TAIL, D), jnp.float32),
        pltpu.SemaphoreType.DMA,
        pltpu.SemaphoreType.DMA,
    ],
)
def _sc_segsum(y_hbm, src_hbm, dst_hbm, zeros_hbm, part_hbm,
               acc_sh, idxs0, idxs1, idxd0, idxd1, idxs_t, idxd_t,
               rows0, rows1, rows_t, sem0, sem1):
    cid = lax.axis_index("c")
    sid = lax.axis_index("s")
    w = cid * NS + sid
    idxs = (idxs0, idxs1)
    idxd = (idxd0, idxd1)
    rows = (rows0, rows1)
    sems = (sem0, sem1)
    pltpu.sync_copy(zeros_hbm, acc_sh.at[pl.ds(sid * RPT, RPT)])
    plsc.subcore_barrier()

    def load_and_gather(c, j):
        base = w * EPW + c * CH
        pltpu.sync_copy(src_hbm.at[pl.ds(base, CH)], idxs[j])
        pltpu.sync_copy(dst_hbm.at[pl.ds(base, CH)], idxd[j])
        pltpu.async_copy(y_hbm.at[idxs[j]], rows[j], sems[j])

    load_and_gather(0, 0)

    def group(g, carry):
        for j in (0, 1):
            c = 2 * g + j
            if j == 0:
                load_and_gather(c + 1, 1)
            else:
                @pl.when(g < NFULL // 2 - 1)
                def _():
                    load_and_gather(c + 1, 0)

            pltpu.make_async_copy(y_hbm.at[idxs[j]], rows[j], sems[j]).wait()
            pltpu.sync_copy(rows[j], acc_sh.at[idxd[j]], add=True)
        return carry

    lax.fori_loop(0, NFULL // 2, group, 0)
    base = w * EPW + NFULL * CH
    pltpu.sync_copy(src_hbm.at[pl.ds(base, TAIL)], idxs_t)
    pltpu.sync_copy(dst_hbm.at[pl.ds(base, TAIL)], idxd_t)
    pltpu.async_copy(y_hbm.at[idxs_t], rows_t, sem0).wait()
    pltpu.sync_copy(rows_t, acc_sh.at[idxd_t], add=True)
    plsc.subcore_barrier()
    pltpu.sync_copy(acc_sh.at[pl.ds(sid * RPT, RPT)],
                    part_hbm.at[cid, pl.ds(sid * RPT, RPT)])


_BLK = 1000
_GRID = N // _BLK


def _dis_of(d0, d1):
    deg = d0[0, :, 0:1] + d1[0, :, 0:1] + 1.0
    return lax.rsqrt(deg)


def _tc_scale_body(d0, d1, x_ref, y_ref):
    y_ref[...] = _dis_of(d0, d1) * x_ref[...]


def _gate_of(top_ref, g_ref):
    logits = lax.dot_general(top_ref[...], g_ref[...], (((1,), (1,)), ((), ())),
                             precision=_HIGH) * (1.0 / 101.0)
    m = jnp.max(logits, axis=-1, keepdims=True)
    e = jnp.exp(logits - m)
    return e / jnp.sum(e, axis=-1, keepdims=True)


def _moe_of(agg, gate, w_ref, b_ref):
    out = jnp.zeros(agg.shape, agg.dtype)
    for i in range(3):
        eo = lax.dot_general(agg, w_ref[i], (((1,), (1,)), ((), ())),
                             precision=_HIGH) + b_ref[i][None, :]
        out = out + gate[:, i][:, None] * jnp.maximum(eo, 0.0)
    return out


def _tc_layer_body(s0, s1, d0, d1, x_ref, top_ref, w_ref, b_ref, g_ref,
                   h_ref, y2_ref):
    dis = _dis_of(d0, d1)
    agg = dis * (s0[0] + s1[0]) + (dis * dis) * x_ref[...]
    gate = _gate_of(top_ref, g_ref)
    h = _moe_of(agg, gate, w_ref, b_ref)
    h_ref[...] = h
    y2_ref[...] = dis * h


def _tc_final_body(s0, s1, d0, d1, h1_ref, top_ref, w_ref, b_ref, g_ref,
                   fcw_ref, fcb_ref, out_ref):
    dis = _dis_of(d0, d1)
    agg = dis * (s0[0] + s1[0]) + (dis * dis) * h1_ref[...]
    gate = _gate_of(top_ref, g_ref)
    h2 = _moe_of(agg, gate, w_ref, b_ref)
    out_ref[...] = lax.dot_general(h2, fcw_ref[...], (((1,), (1,)), ((), ())),
                                   precision=_HIGH) + fcb_ref[...][None, :]


def _half_specs(width):
    return [
        pl.BlockSpec((1, _BLK, width), lambda i: (0, i, 0)),
        pl.BlockSpec((1, _BLK, width), lambda i: (1, i, 0)),
    ]


def _full(shape):
    nd = len(shape)
    return pl.BlockSpec(shape, lambda i, _nd=nd: (0,) * _nd)


_tc_scale = pl.pallas_call(
    _tc_scale_body,
    grid=(_GRID,),
    in_specs=_half_specs(DEGW) + [pl.BlockSpec((_BLK, D), lambda i: (i, 0))],
    out_specs=pl.BlockSpec((_BLK, D), lambda i: (i, 0)),
    out_shape=jax.ShapeDtypeStruct((N, D), jnp.float32),
)

_tc_layer = pl.pallas_call(
    _tc_layer_body,
    grid=(_GRID,),
    in_specs=(
        _half_specs(D) + _half_specs(DEGW)
        + [pl.BlockSpec((_BLK, D), lambda i: (i, 0)),
           pl.BlockSpec((_BLK, 4), lambda i: (i, 0)),
           _full((3, D, D)), _full((3, D)), _full((3, 4))]
    ),
    out_specs=[pl.BlockSpec((_BLK, D), lambda i: (i, 0))] * 2,
    out_shape=[jax.ShapeDtypeStruct((N, D), jnp.float32)] * 2,
)

_tc_final = pl.pallas_call(
    _tc_final_body,
    grid=(_GRID,),
    in_specs=(
        _half_specs(D) + _half_specs(DEGW)
        + [pl.BlockSpec((_BLK, D), lambda i: (i, 0)),
           pl.BlockSpec((_BLK, 4), lambda i: (i, 0)),
           _full((3, D, D)), _full((3, D)), _full((3, 4)),
           _full((D, D)), _full((D,))]
    ),
    out_specs=pl.BlockSpec((_BLK, D), lambda i: (i, 0)),
    out_shape=jax.ShapeDtypeStruct((N, D), jnp.float32),
)


def kernel(x, edge_index, top_features, W1, b1, W2, b2, G1, G2, fcW, fcb):
    src = edge_index[0]
    dst = edge_index[1]
    ones_rows = jnp.ones((CH, D), jnp.float32)
    zeros_rows = jnp.zeros((RPT, D), jnp.float32)

    degp = _sc_degree(dst, ones_rows, zeros_rows)
    y1 = _tc_scale(degp, degp, x)
    s1p = _sc_segsum(y1, src, dst, zeros_rows)
    h1, y2 = _tc_layer(s1p, s1p, degp, degp, x, top_features, W1, b1, G1)
    s2p = _sc_segsum(y2, src, dst, zeros_rows)
    return _tc_final(s2p, s2p, degp, degp, h1, top_features,
                     W2, b2, G2, fcW, fcb)


# speedup vs baseline: 36.1026x; 1.0009x over previous
import functools

import jax
import jax.numpy as jnp
from jax import lax
from jax.experimental import pallas as pl
from jax.experimental.pallas import tpu as pltpu
from jax.experimental.pallas import tpu_sc as plsc

N = 10000
E = 320000
D = 128
NC = 2
NS = 16
NW = NC * NS
EPW = E // NW
CH = 128
NFULL = EPW // CH
TAIL = EPW - NFULL * CH
NP = 10240
RPT = NP // NS
DEGW = D

_MESH = plsc.VectorSubcoreMesh(core_axis_name="c", subcore_axis_name="s")
_HIGH = jax.lax.Precision.HIGHEST


@functools.partial(
    pl.kernel,
    out_type=jax.ShapeDtypeStruct((NC, NP, D), jnp.float32),
    mesh=_MESH,
    scratch_types=[
        pltpu.VMEM_SHARED((NP, D), jnp.float32),
        pltpu.VMEM((CH,), jnp.int32),
        pltpu.VMEM((CH,), jnp.int32),
        pltpu.VMEM((TAIL,), jnp.int32),
        pltpu.VMEM((CH, D), jnp.float32),
        pltpu.SemaphoreType.DMA,
        pltpu.SemaphoreType.DMA,
    ],
)
def _sc_degree(dst_hbm, ones_hbm, zeros_hbm, degp_hbm,
               acc_sh, idx0_v, idx1_v, idxt_v, ones_v, sem0, sem1):
    cid = lax.axis_index("c")
    sid = lax.axis_index("s")
    w = cid * NS + sid
    idx = (idx0_v, idx1_v)
    sems = (sem0, sem1)
    pltpu.sync_copy(zeros_hbm, acc_sh.at[pl.ds(sid * RPT, RPT)])
    pltpu.sync_copy(ones_hbm, ones_v)
    plsc.subcore_barrier()

    def group(g, carry):
        for j in (0, 1):
            c = 2 * g + j

            @pl.when(g > 0)
            def _():
                pltpu.make_async_copy(ones_v, acc_sh.at[idx[j]],
                                      sems[j]).wait()

            pltpu.sync_copy(dst_hbm.at[pl.ds(w * EPW + c * CH, CH)], idx[j])
            pltpu.async_copy(ones_v, acc_sh.at[idx[j]], sems[j], add=True)
        return carry

    lax.fori_loop(0, NFULL // 2, group, 0)
    for j in (0, 1):
        pltpu.make_async_copy(ones_v, acc_sh.at[idx[j]], sems[j]).wait()
    base = w * EPW + NFULL * CH
    pltpu.sync_copy(dst_hbm.at[pl.ds(base, TAIL)], idxt_v)
    pltpu.sync_copy(ones_v.at[pl.ds(0, TAIL)], acc_sh.at[idxt_v], add=True)
    plsc.subcore_barrier()
    pltpu.sync_copy(acc_sh.at[pl.ds(sid * RPT, RPT)],
                    degp_hbm.at[cid, pl.ds(sid * RPT, RPT)])


@functools.partial(
    pl.kernel,
    out_type=jax.ShapeDtypeStruct((NC, NP, D), jnp.float32),
    mesh=_MESH,
    scratch_types=[
        pltpu.VMEM_SHARED((NP, D), jnp.float32),
        pltpu.VMEM((CH,), jnp.int32),
        pltpu.VMEM((CH,), jnp.int32),
        pltpu.VMEM((CH,), jnp.int32),
        pltpu.VMEM((CH,), jnp.int32),
        pltpu.VMEM((TAIL,), jnp.int32),
        pltpu.VMEM((TAIL,), jnp.int32),
        pltpu.VMEM((CH, D), jnp.float32),
        pltpu.VMEM((CH, D), jnp.float32),
        pltpu.VMEM((TAIL, D), jnp.float32),
        pltpu.SemaphoreType.DMA,
        pltpu.SemaphoreType.DMA,
        pltpu.SemaphoreType.DMA,
        pltpu.SemaphoreType.DMA,
    ],
)
def _sc_segsum(y_hbm, src_hbm, dst_hbm, zeros_hbm, part_hbm,
               acc_sh, is0, is1, id0, id1, idxs_t, idxd_t,
               rows0, rows1, rows_t, g0, g1, s0, s1):
    cid = lax.axis_index("c")
    sid = lax.axis_index("s")
    w = cid * NS + sid
    rows = (rows0, rows1)
    gsem = (g0, g1)
    ssem = (s0, s1)
    idxs = (is0, is1)
    idxd = (id0, id1)
    pltpu.sync_copy(zeros_hbm, acc_sh.at[pl.ds(sid * RPT, RPT)])
    plsc.subcore_barrier()

    def load_and_gather(c, b):
        base = w * EPW + c * CH
        pltpu.sync_copy(src_hbm.at[pl.ds(base, CH)], idxs[b])
        pltpu.sync_copy(dst_hbm.at[pl.ds(base, CH)], idxd[b])
        pltpu.async_copy(y_hbm.at[idxs[b]], rows[b], gsem[b])

    def wait_gather(b):
        pltpu.make_async_copy(y_hbm.at[idxs[b]], rows[b], gsem[b]).wait()

    def wait_scatter(b):
        pltpu.make_async_copy(rows[b], acc_sh.at[idxd[b]], ssem[b]).wait()

    load_and_gather(0, 0)

    def group(g, carry):
        for j in (0, 1):
            c = 2 * g + j
            b, o = j, 1 - j
            if j == 0:
                @pl.when(g > 0)
                def _():
                    wait_scatter(o)
                load_and_gather(c + 1, o)
            else:
                @pl.when(g < NFULL // 2 - 1)
                def _():
                    wait_scatter(o)
                    load_and_gather(c + 1, o)

            wait_gather(b)
            pltpu.async_copy(rows[b], acc_sh.at[idxd[b]], ssem[b], add=True)
        return carry

    lax.fori_loop(0, NFULL // 2, group, 0)
    for j in (0, 1):
        wait_scatter(j)
    base = w * EPW + NFULL * CH
    pltpu.sync_copy(src_hbm.at[pl.ds(base, TAIL)], idxs_t)
    pltpu.sync_copy(dst_hbm.at[pl.ds(base, TAIL)], idxd_t)
    pltpu.async_copy(y_hbm.at[idxs_t], rows_t, g0).wait()
    pltpu.sync_copy(rows_t, acc_sh.at[idxd_t], add=True)
    plsc.subcore_barrier()
    pltpu.sync_copy(acc_sh.at[pl.ds(sid * RPT, RPT)],
                    part_hbm.at[cid, pl.ds(sid * RPT, RPT)])


_BLK = 1000
_GRID = N // _BLK


def _dis_of(d0, d1):
    deg = d0[0, :, 0:1] + d1[0, :, 0:1] + 1.0
    return lax.rsqrt(deg)


def _tc_scale_body(d0, d1, x_ref, y_ref):
    y_ref[...] = _dis_of(d0, d1) * x_ref[...]


def _gate_of(top_ref, g_ref):
    logits = lax.dot_general(top_ref[...], g_ref[...], (((1,), (1,)), ((), ())),
                             precision=_HIGH) * (1.0 / 101.0)
    m = jnp.max(logits, axis=-1, keepdims=True)
    e = jnp.exp(logits - m)
    return e / jnp.sum(e, axis=-1, keepdims=True)


def _moe_of(agg, gate, w_ref, b_ref):
    out = jnp.zeros(agg.shape, agg.dtype)
    for i in range(3):
        eo = lax.dot_general(agg, w_ref[i], (((1,), (1,)), ((), ())),
                             precision=_HIGH) + b_ref[i][None, :]
        out = out + gate[:, i][:, None] * jnp.maximum(eo, 0.0)
    return out


def _tc_layer_body(s0, s1, d0, d1, x_ref, top_ref, w_ref, b_ref, g_ref,
                   h_ref, y2_ref):
    dis = _dis_of(d0, d1)
    agg = dis * (s0[0] + s1[0]) + (dis * dis) * x_ref[...]
    gate = _gate_of(top_ref, g_ref)
    h = _moe_of(agg, gate, w_ref, b_ref)
    h_ref[...] = h
    y2_ref[...] = dis * h


def _tc_final_body(s0, s1, d0, d1, h1_ref, top_ref, w_ref, b_ref, g_ref,
                   fcw_ref, fcb_ref, out_ref):
    dis = _dis_of(d0, d1)
    agg = dis * (s0[0] + s1[0]) + (dis * dis) * h1_ref[...]
    gate = _gate_of(top_ref, g_ref)
    h2 = _moe_of(agg, gate, w_ref, b_ref)
    out_ref[...] = lax.dot_general(h2, fcw_ref[...], (((1,), (1,)), ((), ())),
                                   precision=_HIGH) + fcb_ref[...][None, :]


def _half_specs(width):
    return [
        pl.BlockSpec((1, _BLK, width), lambda i: (0, i, 0)),
        pl.BlockSpec((1, _BLK, width), lambda i: (1, i, 0)),
    ]


def _full(shape):
    nd = len(shape)
    return pl.BlockSpec(shape, lambda i, _nd=nd: (0,) * _nd)


_tc_scale = pl.pallas_call(
    _tc_scale_body,
    grid=(_GRID,),
    in_specs=_half_specs(DEGW) + [pl.BlockSpec((_BLK, D), lambda i: (i, 0))],
    out_specs=pl.BlockSpec((_BLK, D), lambda i: (i, 0)),
    out_shape=jax.ShapeDtypeStruct((N, D), jnp.float32),
)

_tc_layer = pl.pallas_call(
    _tc_layer_body,
    grid=(_GRID,),
    in_specs=(
        _half_specs(D) + _half_specs(DEGW)
        + [pl.BlockSpec((_BLK, D), lambda i: (i, 0)),
           pl.BlockSpec((_BLK, 4), lambda i: (i, 0)),
           _full((3, D, D)), _full((3, D)), _full((3, 4))]
    ),
    out_specs=[pl.BlockSpec((_BLK, D), lambda i: (i, 0))] * 2,
    out_shape=[jax.ShapeDtypeStruct((N, D), jnp.float32)] * 2,
)

_tc_final = pl.pallas_call(
    _tc_final_body,
    grid=(_GRID,),
    in_specs=(
        _half_specs(D) + _half_specs(DEGW)
        + [pl.BlockSpec((_BLK, D), lambda i: (i, 0)),
           pl.BlockSpec((_BLK, 4), lambda i: (i, 0)),
           _full((3, D, D)), _full((3, D)), _full((3, 4)),
           _full((D, D)), _full((D,))]
    ),
    out_specs=pl.BlockSpec((_BLK, D), lambda i: (i, 0)),
    out_shape=jax.ShapeDtypeStruct((N, D), jnp.float32),
)


def kernel(x, edge_index, top_features, W1, b1, W2, b2, G1, G2, fcW, fcb):
    src = edge_index[0]
    dst = edge_index[1]
    ones_rows = jnp.ones((CH, D), jnp.float32)
    zeros_rows = jnp.zeros((RPT, D), jnp.float32)

    degp = _sc_degree(dst, ones_rows, zeros_rows)
    y1 = _tc_scale(degp, degp, x)
    s1p = _sc_segsum(y1, src, dst, zeros_rows)
    h1, y2 = _tc_layer(s1p, s1p, degp, degp, x, top_features, W1, b1, G1)
    s2p = _sc_segsum(y2, src, dst, zeros_rows)
    return _tc_final(s2p, s2p, degp, degp, h1, top_features,
                     W2, b2, G2, fcW, fcb)


# speedup vs baseline: 42.7504x; 1.1841x over previous
import functools

import jax
import jax.numpy as jnp
from jax import lax
from jax.experimental import pallas as pl
from jax.experimental.pallas import tpu as pltpu
from jax.experimental.pallas import tpu_sc as plsc

N = 10000
E = 320000
D = 128
NC = 2
NS = 16
NW = NC * NS
CH = 128
NCHK = E // CH
KB = NCHK // NW
XW = NCHK - KB * NW
NP = 10240
RPT = NP // NS
DEGW = D

_MESH = plsc.VectorSubcoreMesh(core_axis_name="c", subcore_axis_name="s")
_HIGH = jax.lax.Precision.HIGHEST


@functools.partial(
    pl.kernel,
    out_type=jax.ShapeDtypeStruct((NC, NP, D), jnp.float32),
    mesh=_MESH,
    scratch_types=[
        pltpu.VMEM_SHARED((NP, D), jnp.float32),
        pltpu.VMEM((CH,), jnp.int32),
        pltpu.VMEM((CH,), jnp.int32),
        pltpu.VMEM((CH, D), jnp.float32),
        pltpu.SemaphoreType.DMA,
        pltpu.SemaphoreType.DMA,
        pltpu.SemaphoreType.DMA,
        pltpu.SemaphoreType.DMA,
    ],
)
def _sc_degree(dst_hbm, ones_hbm, zeros_hbm, degp_hbm,
               acc_sh, id0, id1, ones_v, d0, d1, s0, s1):
    cid = lax.axis_index("c")
    sid = lax.axis_index("s")
    w = cid * NS + sid
    cs = w * KB + jnp.minimum(w, XW)
    idxd = (id0, id1)
    dsem = (d0, d1)
    ssem = (s0, s1)
    pltpu.sync_copy(zeros_hbm, acc_sh.at[pl.ds(sid * RPT, RPT)])
    pltpu.sync_copy(ones_hbm, ones_v)
    plsc.subcore_barrier()

    def prefetch(c, b):
        pltpu.async_copy(dst_hbm.at[pl.ds((cs + c) * CH, CH)], idxd[b],
                         dsem[b])

    def wait_load(b):
        pltpu.make_async_copy(dst_hbm.at[pl.ds(0, CH)], idxd[b],
                              dsem[b]).wait()

    def wait_scatter(b):
        pltpu.make_async_copy(ones_v, acc_sh.at[idxd[b]], ssem[b]).wait()

    prefetch(0, 0)

    def group(g, carry):
        for j in (0, 1):
            c = 2 * g + j
            b, o = j, 1 - j
            if j == 0:
                @pl.when(g > 0)
                def _():
                    wait_scatter(o)
                prefetch(c + 1, o)
            else:
                @pl.when(g < KB // 2 - 1)
                def _():
                    wait_scatter(o)
                    prefetch(c + 1, o)

            wait_load(b)
            pltpu.async_copy(ones_v, acc_sh.at[idxd[b]], ssem[b], add=True)
        return carry

    lax.fori_loop(0, KB // 2, group, 0)
    for b in (0, 1):
        wait_scatter(b)

    @pl.when(w < XW)
    def _():
        prefetch(KB, 0)
        wait_load(0)
        pltpu.sync_copy(ones_v, acc_sh.at[id0], add=True)

    plsc.subcore_barrier()
    pltpu.sync_copy(acc_sh.at[pl.ds(sid * RPT, RPT)],
                    degp_hbm.at[cid, pl.ds(sid * RPT, RPT)])


@functools.partial(
    pl.kernel,
    out_type=jax.ShapeDtypeStruct((NC, NP, D), jnp.float32),
    mesh=_MESH,
    scratch_types=[
        pltpu.VMEM_SHARED((NP, D), jnp.float32),
        pltpu.VMEM(((KB + 1) * CH,), jnp.int32),
        pltpu.VMEM((CH,), jnp.int32),
        pltpu.VMEM((CH,), jnp.int32),
        pltpu.VMEM((CH, D), jnp.float32),
        pltpu.VMEM((CH, D), jnp.float32),
        pltpu.SemaphoreType.DMA,
        pltpu.SemaphoreType.DMA,
        pltpu.SemaphoreType.DMA,
        pltpu.SemaphoreType.DMA,
    ],
)
def _sc_segsum(y_hbm, src_hbm, dst_hbm, zeros_hbm, part_hbm,
               acc_sh, stag_s, id0, id1, rows0, rows1, g0, g1, s0, s1):
    cid = lax.axis_index("c")
    sid = lax.axis_index("s")
    w = cid * NS + sid
    cs = w * KB + jnp.minimum(w, XW)
    rows = (rows0, rows1)
    idxd = (id0, id1)
    gsem = (g0, g1)
    ssem = (s0, s1)
    pltpu.sync_copy(zeros_hbm, acc_sh.at[pl.ds(sid * RPT, RPT)])
    pltpu.sync_copy(src_hbm.at[pl.ds(cs * CH, KB * CH)],
                    stag_s.at[pl.ds(0, KB * CH)])

    @pl.when(w < XW)
    def _():
        pltpu.sync_copy(src_hbm.at[pl.ds((cs + KB) * CH, CH)],
                        stag_s.at[pl.ds(KB * CH, CH)])

    plsc.subcore_barrier()

    def prefetch(c, b):
        pltpu.async_copy(dst_hbm.at[pl.ds((cs + c) * CH, CH)], idxd[b],
                         gsem[b])
        pltpu.async_copy(y_hbm.at[stag_s.at[pl.ds(c * CH, CH)]], rows[b],
                         gsem[b])

    def wait_slot(b):
        pltpu.make_async_copy(dst_hbm.at[pl.ds(0, CH)], idxd[b],
                              gsem[b]).wait()
        pltpu.make_async_copy(y_hbm.at[stag_s.at[pl.ds(0, CH)]], rows[b],
                              gsem[b]).wait()

    def wait_scatter(b):
        pltpu.make_async_copy(rows[b], acc_sh.at[idxd[b]], ssem[b]).wait()

    prefetch(0, 0)

    def group(g, carry):
        for j in (0, 1):
            c = 2 * g + j
            b, o = j, 1 - j
            if j == 0:
                @pl.when(g > 0)
                def _():
                    wait_scatter(o)
                prefetch(c + 1, o)
            else:
                @pl.when(g < KB // 2 - 1)
                def _():
                    wait_scatter(o)
                    prefetch(c + 1, o)

            wait_slot(b)
            pltpu.async_copy(rows[b], acc_sh.at[idxd[b]], ssem[b], add=True)
        return carry

    lax.fori_loop(0, KB // 2, group, 0)
    for b in (0, 1):
        wait_scatter(b)

    @pl.when(w < XW)
    def _():
        prefetch(KB, 0)
        wait_slot(0)
        pltpu.sync_copy(rows0, acc_sh.at[id0], add=True)

    plsc.subcore_barrier()
    pltpu.sync_copy(acc_sh.at[pl.ds(sid * RPT, RPT)],
                    part_hbm.at[cid, pl.ds(sid * RPT, RPT)])


_BLK = 1000
_GRID = N // _BLK


def _dis_of(d0, d1):
    deg = d0[0, :, 0:1] + d1[0, :, 0:1] + 1.0
    return lax.rsqrt(deg)


def _tc_scale_body(d0, d1, x_ref, y_ref):
    y_ref[...] = _dis_of(d0, d1) * x_ref[...]


def _gate_of(top_ref, g_ref):
    logits = lax.dot_general(top_ref[...], g_ref[...], (((1,), (1,)), ((), ())),
                             precision=_HIGH) * (1.0 / 101.0)
    m = jnp.max(logits, axis=-1, keepdims=True)
    e = jnp.exp(logits - m)
    return e / jnp.sum(e, axis=-1, keepdims=True)


def _moe_of(agg, gate, w_ref, b_ref):
    out = jnp.zeros(agg.shape, agg.dtype)
    for i in range(3):
        eo = lax.dot_general(agg, w_ref[i], (((1,), (1,)), ((), ())),
                             precision=_HIGH) + b_ref[i][None, :]
        out = out + gate[:, i][:, None] * jnp.maximum(eo, 0.0)
    return out


def _tc_layer_body(s0, s1, d0, d1, x_ref, top_ref, w_ref, b_ref, g_ref,
                   h_ref, y2_ref):
    dis = _dis_of(d0, d1)
    agg = dis * (s0[0] + s1[0]) + (dis * dis) * x_ref[...]
    gate = _gate_of(top_ref, g_ref)
    h = _moe_of(agg, gate, w_ref, b_ref)
    h_ref[...] = h
    y2_ref[...] = dis * h


def _tc_final_body(s0, s1, d0, d1, h1_ref, top_ref, w_ref, b_ref, g_ref,
                   fcw_ref, fcb_ref, out_ref):
    dis = _dis_of(d0, d1)
    agg = dis * (s0[0] + s1[0]) + (dis * dis) * h1_ref[...]
    gate = _gate_of(top_ref, g_ref)
    h2 = _moe_of(agg, gate, w_ref, b_ref)
    out_ref[...] = lax.dot_general(h2, fcw_ref[...], (((1,), (1,)), ((), ())),
                                   precision=_HIGH) + fcb_ref[...][None, :]


def _half_specs(width):
    return [
        pl.BlockSpec((1, _BLK, width), lambda i: (0, i, 0)),
        pl.BlockSpec((1, _BLK, width), lambda i: (1, i, 0)),
    ]


def _full(shape):
    nd = len(shape)
    return pl.BlockSpec(shape, lambda i, _nd=nd: (0,) * _nd)


_tc_scale = pl.pallas_call(
    _tc_scale_body,
    grid=(_GRID,),
    in_specs=_half_specs(DEGW) + [pl.BlockSpec((_BLK, D), lambda i: (i, 0))],
    out_specs=pl.BlockSpec((_BLK, D), lambda i: (i, 0)),
    out_shape=jax.ShapeDtypeStruct((N, D), jnp.float32),
)

_tc_layer = pl.pallas_call(
    _tc_layer_body,
    grid=(_GRID,),
    in_specs=(
        _half_specs(D) + _half_specs(DEGW)
        + [pl.BlockSpec((_BLK, D), lambda i: (i, 0)),
           pl.BlockSpec((_BLK, 4), lambda i: (i, 0)),
           _full((3, D, D)), _full((3, D)), _full((3, 4))]
    ),
    out_specs=[pl.BlockSpec((_BLK, D), lambda i: (i, 0))] * 2,
    out_shape=[jax.ShapeDtypeStruct((N, D), jnp.float32)] * 2,
)

_tc_final = pl.pallas_call(
    _tc_final_body,
    grid=(_GRID,),
    in_specs=(
        _half_specs(D) + _half_specs(DEGW)
        + [pl.BlockSpec((_BLK, D), lambda i: (i, 0)),
           pl.BlockSpec((_BLK, 4), lambda i: (i, 0)),
           _full((3, D, D)), _full((3, D)), _full((3, 4)),
           _full((D, D)), _full((D,))]
    ),
    out_specs=pl.BlockSpec((_BLK, D), lambda i: (i, 0)),
    out_shape=jax.ShapeDtypeStruct((N, D), jnp.float32),
)


def kernel(x, edge_index, top_features, W1, b1, W2, b2, G1, G2, fcW, fcb):
    src = edge_index[0]
    dst = edge_index[1]
    ones_rows = jnp.ones((CH, D), jnp.float32)
    zeros_rows = jnp.zeros((RPT, D), jnp.float32)

    degp = _sc_degree(dst, ones_rows, zeros_rows)
    y1 = _tc_scale(degp, degp, x)
    s1p = _sc_segsum(y1, src, dst, zeros_rows)
    h1, y2 = _tc_layer(s1p, s1p, degp, degp, x, top_features, W1, b1, G1)
    s2p = _sc_segsum(y2, src, dst, zeros_rows)
    return _tc_final(s2p, s2p, degp, degp, h1, top_features,
                     W2, b2, G2, fcW, fcb)


# speedup vs baseline: 45.4524x; 1.0632x over previous
import functools

import jax
import jax.numpy as jnp
from jax import lax
from jax.experimental import pallas as pl
from jax.experimental.pallas import tpu as pltpu
from jax.experimental.pallas import tpu_sc as plsc

N = 10000
E = 320000
D = 128
NC = 2
NS = 16
NW = NC * NS
CH = 128
NCHK = E // CH
KB = NCHK // NW
XW = NCHK - KB * NW
NP = 10240
RPT = NP // NS
DEGW = D

_MESH = plsc.VectorSubcoreMesh(core_axis_name="c", subcore_axis_name="s")
_HIGH = jax.lax.Precision.HIGHEST


@functools.partial(
    pl.kernel,
    out_type=jax.ShapeDtypeStruct((NC, NP, D), jnp.float32),
    mesh=_MESH,
    scratch_types=[
        pltpu.VMEM_SHARED((NP, D), jnp.float32),
        pltpu.VMEM((CH,), jnp.int32),
        pltpu.VMEM((CH,), jnp.int32),
        pltpu.VMEM((CH, D), jnp.float32),
        pltpu.SemaphoreType.DMA,
        pltpu.SemaphoreType.DMA,
        pltpu.SemaphoreType.DMA,
        pltpu.SemaphoreType.DMA,
    ],
)
def _sc_degree(dst_hbm, ones_hbm, zeros_hbm, degp_hbm,
               acc_sh, id0, id1, ones_v, d0, d1, s0, s1):
    cid = lax.axis_index("c")
    sid = lax.axis_index("s")
    w = cid * NS + sid
    cs = w * KB + jnp.minimum(w, XW)
    idxd = (id0, id1)
    dsem = (d0, d1)
    ssem = (s0, s1)
    pltpu.sync_copy(zeros_hbm, acc_sh.at[pl.ds(sid * RPT, RPT)])
    pltpu.sync_copy(ones_hbm, ones_v)
    plsc.subcore_barrier()

    def prefetch(c, b):
        pltpu.async_copy(dst_hbm.at[pl.ds((cs + c) * CH, CH)], idxd[b],
                         dsem[b])

    def wait_load(b):
        pltpu.make_async_copy(dst_hbm.at[pl.ds(0, CH)], idxd[b],
                              dsem[b]).wait()

    def wait_scatter(b):
        pltpu.make_async_copy(ones_v, acc_sh.at[idxd[b]], ssem[b]).wait()

    prefetch(0, 0)

    def group(g, carry):
        for j in (0, 1):
            c = 2 * g + j
            b, o = j, 1 - j
            if j == 0:
                @pl.when(g > 0)
                def _():
                    wait_scatter(o)
                prefetch(c + 1, o)
            else:
                @pl.when(g < KB // 2 - 1)
                def _():
                    wait_scatter(o)
                    prefetch(c + 1, o)

            wait_load(b)
            pltpu.async_copy(ones_v, acc_sh.at[idxd[b]], ssem[b], add=True)
        return carry

    lax.fori_loop(0, KB // 2, group, 0)
    for b in (0, 1):
        wait_scatter(b)

    @pl.when(w < XW)
    def _():
        prefetch(KB, 0)
        wait_load(0)
        pltpu.sync_copy(ones_v, acc_sh.at[id0], add=True)

    plsc.subcore_barrier()
    pltpu.sync_copy(acc_sh.at[pl.ds(sid * RPT, RPT)],
                    degp_hbm.at[cid, pl.ds(sid * RPT, RPT)])


@functools.partial(
    pl.kernel,
    out_type=jax.ShapeDtypeStruct((NC, NP, D), jnp.float32),
    mesh=_MESH,
    scratch_types=[
        pltpu.VMEM_SHARED((NP, D), jnp.float32),
        pltpu.VMEM(((KB + 1) * CH,), jnp.int32),
        pltpu.VMEM((CH,), jnp.int32),
        pltpu.VMEM((CH,), jnp.int32),
        pltpu.VMEM((CH, D), jnp.float32),
        pltpu.VMEM((CH, D), jnp.float32),
        pltpu.SemaphoreType.DMA,
        pltpu.SemaphoreType.DMA,
        pltpu.SemaphoreType.DMA,
        pltpu.SemaphoreType.DMA,
    ],
)
def _sc_segsum(y_hbm, src_hbm, dst_hbm, zeros_hbm, part_hbm,
               acc_sh, stag_s, id0, id1, rows0, rows1, g0, g1, s0, s1):
    cid = lax.axis_index("c")
    sid = lax.axis_index("s")
    w = cid * NS + sid
    cs = w * KB + jnp.minimum(w, XW)
    rows = (rows0, rows1)
    idxd = (id0, id1)
    gsem = (g0, g1)
    ssem = (s0, s1)
    pltpu.sync_copy(zeros_hbm, acc_sh.at[pl.ds(sid * RPT, RPT)])
    pltpu.sync_copy(src_hbm.at[pl.ds(cs * CH, KB * CH)],
                    stag_s.at[pl.ds(0, KB * CH)])

    @pl.when(w < XW)
    def _():
        pltpu.sync_copy(src_hbm.at[pl.ds((cs + KB) * CH, CH)],
                        stag_s.at[pl.ds(KB * CH, CH)])

    plsc.subcore_barrier()

    def prefetch(c, b):
        pltpu.async_copy(dst_hbm.at[pl.ds((cs + c) * CH, CH)], idxd[b],
                         gsem[b])
        pltpu.async_copy(y_hbm.at[stag_s.at[pl.ds(c * CH, CH)]], rows[b],
                         gsem[b])

    def wait_slot(b):
        pltpu.make_async_copy(dst_hbm.at[pl.ds(0, CH)], idxd[b],
                              gsem[b]).wait()
        pltpu.make_async_copy(y_hbm.at[stag_s.at[pl.ds(0, CH)]], rows[b],
                              gsem[b]).wait()

    def wait_scatter(b):
        pltpu.make_async_copy(rows[b], acc_sh.at[idxd[b]], ssem[b]).wait()

    prefetch(0, 0)

    def group(g, carry):
        for j in (0, 1):
            c = 2 * g + j
            b, o = j, 1 - j
            if j == 0:
                @pl.when(g > 0)
                def _():
                    wait_scatter(o)
                prefetch(c + 1, o)
            else:
                @pl.when(g < KB // 2 - 1)
                def _():
                    wait_scatter(o)
                    prefetch(c + 1, o)

            wait_slot(b)
            pltpu.async_copy(rows[b], acc_sh.at[idxd[b]], ssem[b], add=True)
        return carry

    lax.fori_loop(0, KB // 2, group, 0)
    for b in (0, 1):
        wait_scatter(b)

    @pl.when(w < XW)
    def _():
        prefetch(KB, 0)
        wait_slot(0)
        pltpu.sync_copy(rows0, acc_sh.at[id0], add=True)

    plsc.subcore_barrier()
    pltpu.sync_copy(acc_sh.at[pl.ds(sid * RPT, RPT)],
                    part_hbm.at[cid, pl.ds(sid * RPT, RPT)])


_BLK = 2000
_GRID = N // _BLK


def _dis_of(d0, d1):
    deg = d0[0, :, 0:1] + d1[0, :, 0:1] + 1.0
    return lax.rsqrt(deg)


def _tc_scale_body(d0, d1, x_ref, y_ref):
    y_ref[...] = _dis_of(d0, d1) * x_ref[...]


def _gate_of(top_ref, g_ref):
    logits = lax.dot_general(top_ref[...], g_ref[...], (((1,), (1,)), ((), ())),
                             precision=_HIGH) * (1.0 / 101.0)
    m = jnp.max(logits, axis=-1, keepdims=True)
    e = jnp.exp(logits - m)
    return e / jnp.sum(e, axis=-1, keepdims=True)


def _moe_of(agg, gate, wc_ref, bc_ref):
    eo = lax.dot_general(agg, wc_ref[...], (((1,), (0,)), ((), ())),
                         precision=_HIGH) + bc_ref[...][None, :]
    eo = jnp.maximum(eo, 0.0)
    out = gate[:, 0][:, None] * eo[:, 0:D]
    out = out + gate[:, 1][:, None] * eo[:, D:2 * D]
    out = out + gate[:, 2][:, None] * eo[:, 2 * D:3 * D]
    return out


def _tc_layer_body(s0, s1, d0, d1, x_ref, top_ref, w_ref, b_ref, g_ref,
                   h_ref, y2_ref):
    dis = _dis_of(d0, d1)
    agg = dis * (s0[0] + s1[0]) + (dis * dis) * x_ref[...]
    gate = _gate_of(top_ref, g_ref)
    h = _moe_of(agg, gate, w_ref, b_ref)
    h_ref[...] = h
    y2_ref[...] = dis * h


def _tc_final_body(s0, s1, d0, d1, h1_ref, top_ref, w_ref, b_ref, g_ref,
                   fcw_ref, fcb_ref, out_ref):
    dis = _dis_of(d0, d1)
    agg = dis * (s0[0] + s1[0]) + (dis * dis) * h1_ref[...]
    gate = _gate_of(top_ref, g_ref)
    h2 = _moe_of(agg, gate, w_ref, b_ref)
    out_ref[...] = lax.dot_general(h2, fcw_ref[...], (((1,), (1,)), ((), ())),
                                   precision=_HIGH) + fcb_ref[...][None, :]


def _half_specs(width):
    return [
        pl.BlockSpec((1, _BLK, width), lambda i: (0, i, 0)),
        pl.BlockSpec((1, _BLK, width), lambda i: (1, i, 0)),
    ]


def _full(shape):
    nd = len(shape)
    return pl.BlockSpec(shape, lambda i, _nd=nd: (0,) * _nd)


_tc_scale = pl.pallas_call(
    _tc_scale_body,
    grid=(_GRID,),
    in_specs=_half_specs(DEGW) + [pl.BlockSpec((_BLK, D), lambda i: (i, 0))],
    out_specs=pl.BlockSpec((_BLK, D), lambda i: (i, 0)),
    out_shape=jax.ShapeDtypeStruct((N, D), jnp.float32),
)

_tc_layer = pl.pallas_call(
    _tc_layer_body,
    grid=(_GRID,),
    in_specs=(
        _half_specs(D) + _half_specs(DEGW)
        + [pl.BlockSpec((_BLK, D), lambda i: (i, 0)),
           pl.BlockSpec((_BLK, 4), lambda i: (i, 0)),
           _full((D, 3 * D)), _full((3 * D,)), _full((3, 4))]
    ),
    out_specs=[pl.BlockSpec((_BLK, D), lambda i: (i, 0))] * 2,
    out_shape=[jax.ShapeDtypeStruct((N, D), jnp.float32)] * 2,
)

_tc_final = pl.pallas_call(
    _tc_final_body,
    grid=(_GRID,),
    in_specs=(
        _half_specs(D) + _half_specs(DEGW)
        + [pl.BlockSpec((_BLK, D), lambda i: (i, 0)),
           pl.BlockSpec((_BLK, 4), lambda i: (i, 0)),
           _full((D, 3 * D)), _full((3 * D,)), _full((3, 4)),
           _full((D, D)), _full((D,))]
    ),
    out_specs=pl.BlockSpec((_BLK, D), lambda i: (i, 0)),
    out_shape=jax.ShapeDtypeStruct((N, D), jnp.float32),
)


def kernel(x, edge_index, top_features, W1, b1, W2, b2, G1, G2, fcW, fcb):
    src = edge_index[0]
    dst = edge_index[1]
    ones_rows = jnp.ones((CH, D), jnp.float32)
    zeros_rows = jnp.zeros((RPT, D), jnp.float32)

    wc1 = jnp.transpose(W1, (2, 0, 1)).reshape(D, 3 * D)
    wc2 = jnp.transpose(W2, (2, 0, 1)).reshape(D, 3 * D)
    bc1 = b1.reshape(3 * D)
    bc2 = b2.reshape(3 * D)

    degp = _sc_degree(dst, ones_rows, zeros_rows)
    y1 = _tc_scale(degp, degp, x)
    s1p = _sc_segsum(y1, src, dst, zeros_rows)
    h1, y2 = _tc_layer(s1p, s1p, degp, degp, x, top_features, wc1, bc1, G1)
    s2p = _sc_segsum(y2, src, dst, zeros_rows)
    return _tc_final(s2p, s2p, degp, degp, h1, top_features,
                     wc2, bc2, G2, fcW, fcb)


# speedup vs baseline: 45.8536x; 1.0088x over previous
import functools

import jax
import jax.numpy as jnp
from jax import lax
from jax.experimental import pallas as pl
from jax.experimental.pallas import tpu as pltpu
from jax.experimental.pallas import tpu_sc as plsc

N = 10000
E = 320000
D = 128
NC = 2
NS = 16
NW = NC * NS
CH = 128
NCHK = E // CH
KB = NCHK // NW
XW = NCHK - KB * NW
NP = 10240
RPT = NP // NS
DEGW = D

_MESH = plsc.VectorSubcoreMesh(core_axis_name="c", subcore_axis_name="s")
_HIGH = jax.lax.Precision.HIGHEST


@functools.partial(
    pl.kernel,
    out_type=jax.ShapeDtypeStruct((NC, NP, D), jnp.float32),
    mesh=_MESH,
    scratch_types=[
        pltpu.VMEM_SHARED((NP, D), jnp.float32),
        pltpu.VMEM((CH,), jnp.int32),
        pltpu.VMEM((CH,), jnp.int32),
        pltpu.VMEM((CH, D), jnp.float32),
        pltpu.SemaphoreType.DMA,
        pltpu.SemaphoreType.DMA,
        pltpu.SemaphoreType.DMA,
        pltpu.SemaphoreType.DMA,
    ],
)
def _sc_degree(dst_hbm, ones_hbm, zeros_hbm, degp_hbm,
               acc_sh, id0, id1, ones_v, d0, d1, s0, s1):
    cid = lax.axis_index("c")
    sid = lax.axis_index("s")
    w = cid * NS + sid
    cs = w * KB + jnp.minimum(w, XW)
    idxd = (id0, id1)
    dsem = (d0, d1)
    ssem = (s0, s1)
    pltpu.async_copy(zeros_hbm, acc_sh.at[pl.ds(sid * RPT, RPT)], d0)
    pltpu.async_copy(ones_hbm, ones_v, d1)
    pltpu.make_async_copy(zeros_hbm, acc_sh.at[pl.ds(sid * RPT, RPT)],
                          d0).wait()
    pltpu.make_async_copy(ones_hbm, ones_v, d1).wait()
    plsc.subcore_barrier()

    def prefetch(c, b):
        pltpu.async_copy(dst_hbm.at[pl.ds((cs + c) * CH, CH)], idxd[b],
                         dsem[b])

    def wait_load(b):
        pltpu.make_async_copy(dst_hbm.at[pl.ds(0, CH)], idxd[b],
                              dsem[b]).wait()

    def wait_scatter(b):
        pltpu.make_async_copy(ones_v, acc_sh.at[idxd[b]], ssem[b]).wait()

    prefetch(0, 0)

    def group(g, carry):
        for j in (0, 1):
            c = 2 * g + j
            b, o = j, 1 - j
            if j == 0:
                @pl.when(g > 0)
                def _():
                    wait_scatter(o)
                prefetch(c + 1, o)
            else:
                @pl.when(g < KB // 2 - 1)
                def _():
                    wait_scatter(o)
                    prefetch(c + 1, o)

            wait_load(b)
            pltpu.async_copy(ones_v, acc_sh.at[idxd[b]], ssem[b], add=True)
        return carry

    lax.fori_loop(0, KB // 2, group, 0)
    for b in (0, 1):
        wait_scatter(b)

    @pl.when(w < XW)
    def _():
        prefetch(KB, 0)
        wait_load(0)
        pltpu.sync_copy(ones_v, acc_sh.at[id0], add=True)

    plsc.subcore_barrier()
    pltpu.sync_copy(acc_sh.at[pl.ds(sid * RPT, RPT)],
                    degp_hbm.at[cid, pl.ds(sid * RPT, RPT)])


@functools.partial(
    pl.kernel,
    out_type=jax.ShapeDtypeStruct((NC, NP, D), jnp.float32),
    mesh=_MESH,
    scratch_types=[
        pltpu.VMEM_SHARED((NP, D), jnp.float32),
        pltpu.VMEM(((KB + 1) * CH,), jnp.int32),
        pltpu.VMEM((CH,), jnp.int32),
        pltpu.VMEM((CH,), jnp.int32),
        pltpu.VMEM((CH, D), jnp.float32),
        pltpu.VMEM((CH, D), jnp.float32),
        pltpu.SemaphoreType.DMA,
        pltpu.SemaphoreType.DMA,
        pltpu.SemaphoreType.DMA,
        pltpu.SemaphoreType.DMA,
    ],
)
def _sc_segsum(y_hbm, src_hbm, dst_hbm, zeros_hbm, part_hbm,
               acc_sh, stag_s, id0, id1, rows0, rows1, g0, g1, s0, s1):
    cid = lax.axis_index("c")
    sid = lax.axis_index("s")
    w = cid * NS + sid
    cs = w * KB + jnp.minimum(w, XW)
    rows = (rows0, rows1)
    idxd = (id0, id1)
    gsem = (g0, g1)
    ssem = (s0, s1)
    pltpu.async_copy(zeros_hbm, acc_sh.at[pl.ds(sid * RPT, RPT)], g0)
    pltpu.async_copy(src_hbm.at[pl.ds(cs * CH, KB * CH)],
                     stag_s.at[pl.ds(0, KB * CH)], g1)

    @pl.when(w < XW)
    def _():
        pltpu.sync_copy(src_hbm.at[pl.ds((cs + KB) * CH, CH)],
                        stag_s.at[pl.ds(KB * CH, CH)])

    pltpu.make_async_copy(zeros_hbm, acc_sh.at[pl.ds(sid * RPT, RPT)],
                          g0).wait()
    pltpu.make_async_copy(src_hbm.at[pl.ds(cs * CH, KB * CH)],
                          stag_s.at[pl.ds(0, KB * CH)], g1).wait()
    plsc.subcore_barrier()

    def prefetch(c, b):
        pltpu.async_copy(dst_hbm.at[pl.ds((cs + c) * CH, CH)], idxd[b],
                         gsem[b])
        pltpu.async_copy(y_hbm.at[stag_s.at[pl.ds(c * CH, CH)]], rows[b],
                         gsem[b])

    def wait_slot(b):
        pltpu.make_async_copy(dst_hbm.at[pl.ds(0, CH)], idxd[b],
                              gsem[b]).wait()
        pltpu.make_async_copy(y_hbm.at[stag_s.at[pl.ds(0, CH)]], rows[b],
                              gsem[b]).wait()

    def wait_scatter(b):
        pltpu.make_async_copy(rows[b], acc_sh.at[idxd[b]], ssem[b]).wait()

    prefetch(0, 0)

    def group(g, carry):
        for j in (0, 1):
            c = 2 * g + j
            b, o = j, 1 - j
            if j == 0:
                @pl.when(g > 0)
                def _():
                    wait_scatter(o)
                prefetch(c + 1, o)
            else:
                @pl.when(g < KB // 2 - 1)
                def _():
                    wait_scatter(o)
                    prefetch(c + 1, o)

            wait_slot(b)
            pltpu.async_copy(rows[b], acc_sh.at[idxd[b]], ssem[b], add=True)
        return carry

    lax.fori_loop(0, KB // 2, group, 0)
    for b in (0, 1):
        wait_scatter(b)

    @pl.when(w < XW)
    def _():
        prefetch(KB, 0)
        wait_slot(0)
        pltpu.sync_copy(rows0, acc_sh.at[id0], add=True)

    plsc.subcore_barrier()
    pltpu.sync_copy(acc_sh.at[pl.ds(sid * RPT, RPT)],
                    part_hbm.at[cid, pl.ds(sid * RPT, RPT)])


_BLK = 2000
_GRID = N // _BLK


def _dis_of(d0, d1):
    deg = d0[0, :, 0:1] + d1[0, :, 0:1] + 1.0
    return lax.rsqrt(deg)


def _tc_scale_body(d0, d1, x_ref, y_ref):
    y_ref[...] = _dis_of(d0, d1) * x_ref[...]


def _gate_of(top_ref, g_ref):
    logits = lax.dot_general(top_ref[...], g_ref[...], (((1,), (1,)), ((), ())),
                             precision=_HIGH) * (1.0 / 101.0)
    m = jnp.max(logits, axis=-1, keepdims=True)
    e = jnp.exp(logits - m)
    return e / jnp.sum(e, axis=-1, keepdims=True)


def _moe_of(agg, gate, wc_ref, bc_ref):
    eo = lax.dot_general(agg, wc_ref[...], (((1,), (0,)), ((), ())),
                         precision=_HIGH) + bc_ref[...][None, :]
    eo = jnp.maximum(eo, 0.0)
    out = gate[:, 0][:, None] * eo[:, 0:D]
    out = out + gate[:, 1][:, None] * eo[:, D:2 * D]
    out = out + gate[:, 2][:, None] * eo[:, 2 * D:3 * D]
    return out


def _tc_layer_body(s0, s1, d0, d1, x_ref, top_ref, w_ref, b_ref, g_ref,
                   h_ref, y2_ref):
    dis = _dis_of(d0, d1)
    agg = dis * (s0[0] + s1[0]) + (dis * dis) * x_ref[...]
    gate = _gate_of(top_ref, g_ref)
    h = _moe_of(agg, gate, w_ref, b_ref)
    h_ref[...] = h
    y2_ref[...] = dis * h


def _tc_final_body(s0, s1, d0, d1, h1_ref, top_ref, w_ref, b_ref, g_ref,
                   fcw_ref, fcb_ref, out_ref):
    dis = _dis_of(d0, d1)
    agg = dis * (s0[0] + s1[0]) + (dis * dis) * h1_ref[...]
    gate = _gate_of(top_ref, g_ref)
    h2 = _moe_of(agg, gate, w_ref, b_ref)
    out_ref[...] = lax.dot_general(h2, fcw_ref[...], (((1,), (1,)), ((), ())),
                                   precision=_HIGH) + fcb_ref[...][None, :]


def _half_specs(width):
    return [
        pl.BlockSpec((1, _BLK, width), lambda i: (0, i, 0)),
        pl.BlockSpec((1, _BLK, width), lambda i: (1, i, 0)),
    ]


def _full(shape):
    nd = len(shape)
    return pl.BlockSpec(shape, lambda i, _nd=nd: (0,) * _nd)


_tc_scale = pl.pallas_call(
    _tc_scale_body,
    grid=(_GRID,),
    in_specs=_half_specs(DEGW) + [pl.BlockSpec((_BLK, D), lambda i: (i, 0))],
    out_specs=pl.BlockSpec((_BLK, D), lambda i: (i, 0)),
    out_shape=jax.ShapeDtypeStruct((N, D), jnp.float32),
)

_tc_layer = pl.pallas_call(
    _tc_layer_body,
    grid=(_GRID,),
    in_specs=(
        _half_specs(D) + _half_specs(DEGW)
        + [pl.BlockSpec((_BLK, D), lambda i: (i, 0)),
           pl.BlockSpec((_BLK, 4), lambda i: (i, 0)),
           _full((D, 3 * D)), _full((3 * D,)), _full((3, 4))]
    ),
    out_specs=[pl.BlockSpec((_BLK, D), lambda i: (i, 0))] * 2,
    out_shape=[jax.ShapeDtypeStruct((N, D), jnp.float32)] * 2,
)

_tc_final = pl.pallas_call(
    _tc_final_body,
    grid=(_GRID,),
    in_specs=(
        _half_specs(D) + _half_specs(DEGW)
        + [pl.BlockSpec((_BLK, D), lambda i: (i, 0)),
           pl.BlockSpec((_BLK, 4), lambda i: (i, 0)),
           _full((D, 3 * D)), _full((3 * D,)), _full((3, 4)),
           _full((D, D)), _full((D,))]
    ),
    out_specs=pl.BlockSpec((_BLK, D), lambda i: (i, 0)),
    out_shape=jax.ShapeDtypeStruct((N, D), jnp.float32),
)


def kernel(x, edge_index, top_features, W1, b1, W2, b2, G1, G2, fcW, fcb):
    src = edge_index[0]
    dst = edge_index[1]
    ones_rows = jnp.ones((CH, D), jnp.float32)
    zeros_rows = jnp.zeros((RPT, D), jnp.float32)

    wc1 = jnp.transpose(W1, (2, 0, 1)).reshape(D, 3 * D)
    wc2 = jnp.transpose(W2, (2, 0, 1)).reshape(D, 3 * D)
    bc1 = b1.reshape(3 * D)
    bc2 = b2.reshape(3 * D)

    degp = _sc_degree(dst, ones_rows, zeros_rows)
    y1 = _tc_scale(degp, degp, x)
    s1p = _sc_segsum(y1, src, dst, zeros_rows)
    h1, y2 = _tc_layer(s1p, s1p, degp, degp, x, top_features, wc1, bc1, G1)
    s2p = _sc_segsum(y2, src, dst, zeros_rows)
    return _tc_final(s2p, s2p, degp, degp, h1, top_features,
                     wc2, bc2, G2, fcW, fcb)
